# baseline (speedup 1.0000x reference)
import jax
import jax.numpy as jnp
from jax.experimental import pallas as pl

_B, _N, _C = 4, 2048, 64
_K = 5
_CN = 512
_TM = 256
_RT = _N // _TM
_SQRT_C = 8.0


def _cmp(a, b):
    if a is None:
        return (b, None)
    if b is None:
        return (a, None)
    return (jnp.minimum(a, b), jnp.maximum(a, b))


def _oemerge_rec(a, b):
    n = len(a)
    if n == 1:
        return list(_cmp(a[0], b[0]))
    e = _oemerge_rec(a[0::2], b[0::2])
    o = _oemerge_rec(a[1::2], b[1::2])
    out = [e[0]]
    for i in range(n - 1):
        lo, hi = _cmp(o[i], e[i + 1])
        out += [lo, hi]
    out.append(o[n - 1])
    return out


def _oemerge(a, b):
    n = max(len(a), len(b))
    n = 1 << (n - 1).bit_length()
    a = a + [None] * (n - len(a))
    b = b + [None] * (n - len(b))
    return _oemerge_rec(a, b)


def _bottom5(chunks):
    lists = [[c] for c in chunks]
    while len(lists) > 1:
        nxt = []
        for i in range(0, len(lists), 2):
            nxt.append(_oemerge(lists[i], lists[i + 1])[:_K])
        lists = nxt
    return lists[0][:_K]


def _dist_stats_kernel(xr_ref, xa_ref, sqr_ref, sqa_ref, dist_ref, dn_ref, dmax_ref):
    xr = xr_ref[0]
    xa = xa_ref[0]
    sqr = sqr_ref[0, 0][:, None]
    sqa = sqa_ref[0, 0][None, :]
    prod = jax.lax.dot_general(xr, xa, (((1,), (1,)), ((), ())),
                               preferred_element_type=jnp.float32)
    d2 = sqr + sqa - 2.0 * prod
    dist = jnp.sqrt(jnp.maximum(d2, 0.0)) / _SQRT_C
    dist_ref[0] = dist
    dmax_ref[0, 0] = jnp.max(dist, axis=1)

    chunks = [dist[:, i * 128:(i + 1) * 128] for i in range(_N // 128)]
    cand = jnp.concatenate(_bottom5(chunks), axis=1)
    ncand = cand.shape[1]
    col = jax.lax.broadcasted_iota(jnp.int32, (_TM, ncand), 1)
    cur = cand
    for r in range(_K):
        m = jnp.min(cur, axis=1, keepdims=True)
        dn_ref[0, r] = m[:, 0]
        if r < _K - 1:
            first = jnp.min(jnp.where(cur == m, col, ncand), axis=1, keepdims=True)
            cur = jnp.where(col == first, jnp.inf, cur)


def _score_kernel(dist_ref, densr_ref, densa_ref, dmax_ref, score_ref):
    dist = dist_ref[0]
    di = densr_ref[0, 0]
    da = densa_ref[0, 0][None, :]
    dm = dmax_ref[0, 0][:, None]
    masked = jnp.where(da > di[:, None], dist, dm)
    score_ref[0, 0] = jnp.min(masked, axis=1) * di


def _rank_kernel(sa_ref, sr_ref, rank_ref):
    sa = sa_ref[0, 0][None, :]
    si = sr_ref[0, 0][:, None]
    colj = jax.lax.broadcasted_iota(jnp.int32, (_TM, _N), 1)
    rowi = jax.lax.broadcasted_iota(jnp.int32, (_TM, _N), 0) + pl.program_id(1) * _TM
    gt = (sa > si) | ((sa == si) & (colj < rowi))
    rank_ref[0, 0] = jnp.sum(gt.astype(jnp.int32), axis=1)


def _centers_kernel(rank_ref, x_ref, sq_ref, idown_ref, xc_ref, sqc_ref):
    rank = rank_ref[0, 0][None, :]
    r_iota = jax.lax.broadcasted_iota(jnp.int32, (_CN, _N), 0)
    E = (rank == r_iota).astype(jnp.float32)
    i_iota = jax.lax.broadcasted_iota(jnp.int32, (_CN, _N), 1).astype(jnp.float32)
    idown_ref[0, 0] = jnp.sum(E * i_iota, axis=1).astype(jnp.int32)
    xc_ref[0] = jax.lax.dot_general(E, x_ref[0], (((1,), (0,)), ((), ())),
                                    preferred_element_type=jnp.float32)
    sqc_ref[0, 0] = jnp.sum(E * sq_ref[0, 0][None, :], axis=1)


def _assign_kernel(xr_ref, sqr_ref, xc_ref, sqc_ref, rank_ref, ic_ref):
    xr = xr_ref[0]
    sqr = sqr_ref[0, 0][None, :]
    xc = xc_ref[0]
    sqc = sqc_ref[0, 0][:, None]
    prod = jax.lax.dot_general(xc, xr, (((1,), (1,)), ((), ())),
                               preferred_element_type=jnp.float32)
    d2 = sqc + sqr - 2.0 * prod
    distc = jnp.sqrt(jnp.maximum(d2, 0.0)) / _SQRT_C
    minv = jnp.min(distc, axis=0, keepdims=True)
    kio = jax.lax.broadcasted_iota(jnp.int32, (_CN, _TM), 0)
    ic = jnp.min(jnp.where(distc == minv, kio, _CN), axis=0)
    rank = rank_ref[0, 0]
    ic_ref[0, 0] = jnp.where(rank < _CN, rank, ic)


def _merge_kernel(x_ref, ic_ref, xm_ref, nw_ref):
    ic = ic_ref[0, 0]
    kio = jax.lax.broadcasted_iota(jnp.int32, (_CN, _N), 0)
    A = (ic[None, :] == kio).astype(jnp.float32)
    count = jnp.sum(A, axis=1)
    inv = 1.0 / (count + 1e-06)
    nw = jnp.sum(A * inv[:, None], axis=0)
    nw_ref[0, 0] = nw
    xw = x_ref[0] * nw[:, None]
    xm_ref[0] = jax.lax.dot_general(A, xw, (((1,), (0,)), ((), ())),
                                    preferred_element_type=jnp.float32)


def _gather_kernel(it_ref, ic_ref, nw_ref, aw_ref, itn_ref, awn_ref):
    it = it_ref[0, 0]
    icf = ic_ref[0, 0].astype(jnp.float32)[None, :]
    nw = nw_ref[0, 0][None, :]
    mio = jax.lax.broadcasted_iota(jnp.int32, (_TM, _N), 1)
    G = it[:, None] == mio
    itn_ref[0, 0] = jnp.sum(jnp.where(G, icf, 0.0), axis=1).astype(jnp.int32)
    wt = jnp.sum(jnp.where(G, nw, 0.0), axis=1)
    awn_ref[0, 0] = aw_ref[0, 0] * wt


def kernel(x, idx_token, agg_token, agg_weight):
    if agg_weight is None:
        agg_weight = agg_token
    x = x.astype(jnp.float32)
    sq = jnp.sum(x * x, axis=-1)
    sq3 = sq.reshape(_B, 1, _N)

    dist, dn, dmax = pl.pallas_call(
        _dist_stats_kernel,
        grid=(_B, _RT),
        in_specs=[
            pl.BlockSpec((1, _TM, _C), lambda b, t: (b, t, 0)),
            pl.BlockSpec((1, _N, _C), lambda b, t: (b, 0, 0)),
            pl.BlockSpec((1, 1, _TM), lambda b, t: (b, 0, t)),
            pl.BlockSpec((1, 1, _N), lambda b, t: (b, 0, 0)),
        ],
        out_specs=[
            pl.BlockSpec((1, _TM, _N), lambda b, t: (b, t, 0)),
            pl.BlockSpec((1, _K, _TM), lambda b, t: (b, 0, t)),
            pl.BlockSpec((1, 1, _TM), lambda b, t: (b, 0, t)),
        ],
        out_shape=[
            jax.ShapeDtypeStruct((_B, _N, _N), jnp.float32),
            jax.ShapeDtypeStruct((_B, _K, _N), jnp.float32),
            jax.ShapeDtypeStruct((_B, 1, _N), jnp.float32),
        ],
    )(x, x, sq3, sq3)

    dn_t = jnp.transpose(dn, (0, 2, 1))
    dens_flat = jnp.exp(-(dn_t ** 2).mean(axis=-1))
    dens_flat = dens_flat + jax.random.uniform(
        jax.random.key(1), dens_flat.shape, dtype=dens_flat.dtype) * 1e-06
    dens = dens_flat.reshape(_B, 1, _N)

    score = pl.pallas_call(
        _score_kernel,
        grid=(_B, _RT),
        in_specs=[
            pl.BlockSpec((1, _TM, _N), lambda b, t: (b, t, 0)),
            pl.BlockSpec((1, 1, _TM), lambda b, t: (b, 0, t)),
            pl.BlockSpec((1, 1, _N), lambda b, t: (b, 0, 0)),
            pl.BlockSpec((1, 1, _TM), lambda b, t: (b, 0, t)),
        ],
        out_specs=pl.BlockSpec((1, 1, _TM), lambda b, t: (b, 0, t)),
        out_shape=jax.ShapeDtypeStruct((_B, 1, _N), jnp.float32),
    )(dist, dens, dens, dmax)

    rank = pl.pallas_call(
        _rank_kernel,
        grid=(_B, _RT),
        in_specs=[
            pl.BlockSpec((1, 1, _N), lambda b, t: (b, 0, 0)),
            pl.BlockSpec((1, 1, _TM), lambda b, t: (b, 0, t)),
        ],
        out_specs=pl.BlockSpec((1, 1, _TM), lambda b, t: (b, 0, t)),
        out_shape=jax.ShapeDtypeStruct((_B, 1, _N), jnp.int32),
    )(score, score)

    idown, xc, sqc = pl.pallas_call(
        _centers_kernel,
        grid=(_B,),
        in_specs=[
            pl.BlockSpec((1, 1, _N), lambda b: (b, 0, 0)),
            pl.BlockSpec((1, _N, _C), lambda b: (b, 0, 0)),
            pl.BlockSpec((1, 1, _N), lambda b: (b, 0, 0)),
        ],
        out_specs=[
            pl.BlockSpec((1, 1, _CN), lambda b: (b, 0, 0)),
            pl.BlockSpec((1, _CN, _C), lambda b: (b, 0, 0)),
            pl.BlockSpec((1, 1, _CN), lambda b: (b, 0, 0)),
        ],
        out_shape=[
            jax.ShapeDtypeStruct((_B, 1, _CN), jnp.int32),
            jax.ShapeDtypeStruct((_B, _CN, _C), jnp.float32),
            jax.ShapeDtypeStruct((_B, 1, _CN), jnp.float32),
        ],
    )(rank, x, sq3)

    icl = pl.pallas_call(
        _assign_kernel,
        grid=(_B, _RT),
        in_specs=[
            pl.BlockSpec((1, _TM, _C), lambda b, t: (b, t, 0)),
            pl.BlockSpec((1, 1, _TM), lambda b, t: (b, 0, t)),
            pl.BlockSpec((1, _CN, _C), lambda b, t: (b, 0, 0)),
            pl.BlockSpec((1, 1, _CN), lambda b, t: (b, 0, 0)),
            pl.BlockSpec((1, 1, _TM), lambda b, t: (b, 0, t)),
        ],
        out_specs=pl.BlockSpec((1, 1, _TM), lambda b, t: (b, 0, t)),
        out_shape=jax.ShapeDtypeStruct((_B, 1, _N), jnp.int32),
    )(x, sq3, xc, sqc, rank)

    xm, nw = pl.pallas_call(
        _merge_kernel,
        grid=(_B,),
        in_specs=[
            pl.BlockSpec((1, _N, _C), lambda b: (b, 0, 0)),
            pl.BlockSpec((1, 1, _N), lambda b: (b, 0, 0)),
        ],
        out_specs=[
            pl.BlockSpec((1, _CN, _C), lambda b: (b, 0, 0)),
            pl.BlockSpec((1, 1, _N), lambda b: (b, 0, 0)),
        ],
        out_shape=[
            jax.ShapeDtypeStruct((_B, _CN, _C), jnp.float32),
            jax.ShapeDtypeStruct((_B, 1, _N), jnp.float32),
        ],
    )(x, icl)

    it3 = idx_token.reshape(_B, 1, _N)
    aw3 = agg_weight.astype(jnp.float32).reshape(_B, 1, _N)
    itn, awn = pl.pallas_call(
        _gather_kernel,
        grid=(_B, _RT),
        in_specs=[
            pl.BlockSpec((1, 1, _TM), lambda b, t: (b, 0, t)),
            pl.BlockSpec((1, 1, _N), lambda b, t: (b, 0, 0)),
            pl.BlockSpec((1, 1, _N), lambda b, t: (b, 0, 0)),
            pl.BlockSpec((1, 1, _TM), lambda b, t: (b, 0, t)),
        ],
        out_specs=[
            pl.BlockSpec((1, 1, _TM), lambda b, t: (b, 0, t)),
            pl.BlockSpec((1, 1, _TM), lambda b, t: (b, 0, t)),
        ],
        out_shape=[
            jax.ShapeDtypeStruct((_B, 1, _N), jnp.int32),
            jax.ShapeDtypeStruct((_B, 1, _N), jnp.float32),
        ],
    )(it3, icl, nw, aw3)

    return (xm, itn.reshape(_B, _N), awn.reshape(_B, _N, 1),
            icl.reshape(_B, _N), idown.reshape(_B, _CN))


# speedup vs baseline: 1.1296x; 1.1296x over previous
import jax
import jax.numpy as jnp
from jax.experimental import pallas as pl

_B, _N, _C = 4, 2048, 64
_K = 5
_CN = 512
_TM = 512
_RT = _N // _TM
_SQRT_C = 8.0


def _cmp(a, b):
    if a is None:
        return (b, None)
    if b is None:
        return (a, None)
    return (jnp.minimum(a, b), jnp.maximum(a, b))


def _oemerge_rec(a, b):
    n = len(a)
    if n == 1:
        return list(_cmp(a[0], b[0]))
    e = _oemerge_rec(a[0::2], b[0::2])
    o = _oemerge_rec(a[1::2], b[1::2])
    out = [e[0]]
    for i in range(n - 1):
        lo, hi = _cmp(o[i], e[i + 1])
        out += [lo, hi]
    out.append(o[n - 1])
    return out


def _oemerge(a, b):
    n = max(len(a), len(b))
    n = 1 << (n - 1).bit_length()
    a = a + [None] * (n - len(a))
    b = b + [None] * (n - len(b))
    return _oemerge_rec(a, b)


def _bottom5(chunks):
    lists = [[c] for c in chunks]
    while len(lists) > 1:
        nxt = []
        for i in range(0, len(lists), 2):
            nxt.append(_oemerge(lists[i], lists[i + 1])[:_K])
        lists = nxt
    return lists[0][:_K]


def _dist_stats_kernel(xr_ref, xa_ref, sqr_ref, sqa_ref, dist_ref, dn_ref, dmax_ref):
    xr = xr_ref[0]
    xa = xa_ref[0]
    sqr = sqr_ref[0, 0][:, None]
    sqa = sqa_ref[0, 0][None, :]
    prod = jax.lax.dot_general(xr, xa, (((1,), (1,)), ((), ())),
                               preferred_element_type=jnp.float32)
    d2 = sqr + sqa - 2.0 * prod
    dist = jnp.sqrt(jnp.maximum(d2, 0.0)) / _SQRT_C
    dist_ref[0] = dist
    dmax_ref[0, 0] = jnp.max(dist, axis=1)

    chunks = [dist[:, i * 128:(i + 1) * 128] for i in range(_N // 128)]
    cand = jnp.concatenate(_bottom5(chunks), axis=1)
    ncand = cand.shape[1]
    col = jax.lax.broadcasted_iota(jnp.int32, (_TM, ncand), 1)
    cur = cand
    for r in range(_K):
        m = jnp.min(cur, axis=1, keepdims=True)
        dn_ref[0, r] = m[:, 0]
        if r < _K - 1:
            first = jnp.min(jnp.where(cur == m, col, ncand), axis=1, keepdims=True)
            cur = jnp.where(col == first, jnp.inf, cur)


def _score_kernel(dist_ref, densr_ref, densa_ref, dmax_ref, score_ref):
    dist = dist_ref[0]
    di = densr_ref[0, 0]
    da = densa_ref[0, 0][None, :]
    dm = dmax_ref[0, 0][:, None]
    masked = jnp.where(da > di[:, None], dist, dm)
    score_ref[0, 0] = jnp.min(masked, axis=1) * di


def _rank_kernel(sa_ref, sr_ref, rank_ref):
    sa = sa_ref[0, 0][None, :]
    si = sr_ref[0, 0][:, None]
    colj = jax.lax.broadcasted_iota(jnp.int32, (_TM, _N), 1)
    rowi = jax.lax.broadcasted_iota(jnp.int32, (_TM, _N), 0) + pl.program_id(1) * _TM
    gt = (sa > si) | ((sa == si) & (colj < rowi))
    rank_ref[0, 0] = jnp.sum(gt.astype(jnp.int32), axis=1)


def _centers_kernel(rank_ref, x_ref, sq_ref, idown_ref, xc_ref, sqc_ref):
    rank = rank_ref[0, 0][None, :]
    r_iota = jax.lax.broadcasted_iota(jnp.int32, (_CN, _N), 0)
    E = (rank == r_iota).astype(jnp.float32)
    i_iota = jax.lax.broadcasted_iota(jnp.int32, (_CN, _N), 1).astype(jnp.float32)
    idown_ref[0, 0] = jnp.sum(E * i_iota, axis=1).astype(jnp.int32)
    xc_ref[0] = jax.lax.dot_general(E, x_ref[0], (((1,), (0,)), ((), ())),
                                    preferred_element_type=jnp.float32)
    sqc_ref[0, 0] = jnp.sum(E * sq_ref[0, 0][None, :], axis=1)


def _assign_kernel(xr_ref, sqr_ref, xc_ref, sqc_ref, rank_ref, ic_ref):
    xr = xr_ref[0]
    sqr = sqr_ref[0, 0][None, :]
    xc = xc_ref[0]
    sqc = sqc_ref[0, 0][:, None]
    prod = jax.lax.dot_general(xc, xr, (((1,), (1,)), ((), ())),
                               preferred_element_type=jnp.float32)
    d2 = sqc + sqr - 2.0 * prod
    distc = jnp.sqrt(jnp.maximum(d2, 0.0)) / _SQRT_C
    minv = jnp.min(distc, axis=0, keepdims=True)
    kio = jax.lax.broadcasted_iota(jnp.int32, (_CN, _TM), 0)
    ic = jnp.min(jnp.where(distc == minv, kio, _CN), axis=0)
    rank = rank_ref[0, 0]
    ic_ref[0, 0] = jnp.where(rank < _CN, rank, ic)


def _merge_kernel(x_ref, ic_ref, xm_ref, nw_ref):
    ic = ic_ref[0, 0]
    kio = jax.lax.broadcasted_iota(jnp.int32, (_CN, _N), 0)
    A = (ic[None, :] == kio).astype(jnp.float32)
    count = jnp.sum(A, axis=1)
    inv = 1.0 / (count + 1e-06)
    nw = jnp.sum(A * inv[:, None], axis=0)
    nw_ref[0, 0] = nw
    xw = x_ref[0] * nw[:, None]
    xm_ref[0] = jax.lax.dot_general(A, xw, (((1,), (0,)), ((), ())),
                                    preferred_element_type=jnp.float32)


def _gather_kernel(it_ref, ic_ref, nw_ref, aw_ref, itn_ref, awn_ref):
    it = it_ref[0, 0]
    icf = ic_ref[0, 0].astype(jnp.float32)[None, :]
    nw = nw_ref[0, 0][None, :]
    mio = jax.lax.broadcasted_iota(jnp.int32, (_TM, _N), 1)
    G = it[:, None] == mio
    itn_ref[0, 0] = jnp.sum(jnp.where(G, icf, 0.0), axis=1).astype(jnp.int32)
    wt = jnp.sum(jnp.where(G, nw, 0.0), axis=1)
    awn_ref[0, 0] = aw_ref[0, 0] * wt


def kernel(x, idx_token, agg_token, agg_weight):
    if agg_weight is None:
        agg_weight = agg_token
    x = x.astype(jnp.float32)
    sq = jnp.sum(x * x, axis=-1)
    sq3 = sq.reshape(_B, 1, _N)

    dist, dn, dmax = pl.pallas_call(
        _dist_stats_kernel,
        grid=(_B, _RT),
        in_specs=[
            pl.BlockSpec((1, _TM, _C), lambda b, t: (b, t, 0)),
            pl.BlockSpec((1, _N, _C), lambda b, t: (b, 0, 0)),
            pl.BlockSpec((1, 1, _TM), lambda b, t: (b, 0, t)),
            pl.BlockSpec((1, 1, _N), lambda b, t: (b, 0, 0)),
        ],
        out_specs=[
            pl.BlockSpec((1, _TM, _N), lambda b, t: (b, t, 0)),
            pl.BlockSpec((1, _K, _TM), lambda b, t: (b, 0, t)),
            pl.BlockSpec((1, 1, _TM), lambda b, t: (b, 0, t)),
        ],
        out_shape=[
            jax.ShapeDtypeStruct((_B, _N, _N), jnp.float32),
            jax.ShapeDtypeStruct((_B, _K, _N), jnp.float32),
            jax.ShapeDtypeStruct((_B, 1, _N), jnp.float32),
        ],
    )(x, x, sq3, sq3)

    dn_t = jnp.transpose(dn, (0, 2, 1))
    dens_flat = jnp.exp(-(dn_t ** 2).mean(axis=-1))
    dens_flat = dens_flat + jax.random.uniform(
        jax.random.key(1), dens_flat.shape, dtype=dens_flat.dtype) * 1e-06
    dens = dens_flat.reshape(_B, 1, _N)

    score = pl.pallas_call(
        _score_kernel,
        grid=(_B, _RT),
        in_specs=[
            pl.BlockSpec((1, _TM, _N), lambda b, t: (b, t, 0)),
            pl.BlockSpec((1, 1, _TM), lambda b, t: (b, 0, t)),
            pl.BlockSpec((1, 1, _N), lambda b, t: (b, 0, 0)),
            pl.BlockSpec((1, 1, _TM), lambda b, t: (b, 0, t)),
        ],
        out_specs=pl.BlockSpec((1, 1, _TM), lambda b, t: (b, 0, t)),
        out_shape=jax.ShapeDtypeStruct((_B, 1, _N), jnp.float32),
    )(dist, dens, dens, dmax)

    rank = pl.pallas_call(
        _rank_kernel,
        grid=(_B, _RT),
        in_specs=[
            pl.BlockSpec((1, 1, _N), lambda b, t: (b, 0, 0)),
            pl.BlockSpec((1, 1, _TM), lambda b, t: (b, 0, t)),
        ],
        out_specs=pl.BlockSpec((1, 1, _TM), lambda b, t: (b, 0, t)),
        out_shape=jax.ShapeDtypeStruct((_B, 1, _N), jnp.int32),
    )(score, score)

    idown, xc, sqc = pl.pallas_call(
        _centers_kernel,
        grid=(_B,),
        in_specs=[
            pl.BlockSpec((1, 1, _N), lambda b: (b, 0, 0)),
            pl.BlockSpec((1, _N, _C), lambda b: (b, 0, 0)),
            pl.BlockSpec((1, 1, _N), lambda b: (b, 0, 0)),
        ],
        out_specs=[
            pl.BlockSpec((1, 1, _CN), lambda b: (b, 0, 0)),
            pl.BlockSpec((1, _CN, _C), lambda b: (b, 0, 0)),
            pl.BlockSpec((1, 1, _CN), lambda b: (b, 0, 0)),
        ],
        out_shape=[
            jax.ShapeDtypeStruct((_B, 1, _CN), jnp.int32),
            jax.ShapeDtypeStruct((_B, _CN, _C), jnp.float32),
            jax.ShapeDtypeStruct((_B, 1, _CN), jnp.float32),
        ],
    )(rank, x, sq3)

    icl = pl.pallas_call(
        _assign_kernel,
        grid=(_B, _RT),
        in_specs=[
            pl.BlockSpec((1, _TM, _C), lambda b, t: (b, t, 0)),
            pl.BlockSpec((1, 1, _TM), lambda b, t: (b, 0, t)),
            pl.BlockSpec((1, _CN, _C), lambda b, t: (b, 0, 0)),
            pl.BlockSpec((1, 1, _CN), lambda b, t: (b, 0, 0)),
            pl.BlockSpec((1, 1, _TM), lambda b, t: (b, 0, t)),
        ],
        out_specs=pl.BlockSpec((1, 1, _TM), lambda b, t: (b, 0, t)),
        out_shape=jax.ShapeDtypeStruct((_B, 1, _N), jnp.int32),
    )(x, sq3, xc, sqc, rank)

    xm, nw = pl.pallas_call(
        _merge_kernel,
        grid=(_B,),
        in_specs=[
            pl.BlockSpec((1, _N, _C), lambda b: (b, 0, 0)),
            pl.BlockSpec((1, 1, _N), lambda b: (b, 0, 0)),
        ],
        out_specs=[
            pl.BlockSpec((1, _CN, _C), lambda b: (b, 0, 0)),
            pl.BlockSpec((1, 1, _N), lambda b: (b, 0, 0)),
        ],
        out_shape=[
            jax.ShapeDtypeStruct((_B, _CN, _C), jnp.float32),
            jax.ShapeDtypeStruct((_B, 1, _N), jnp.float32),
        ],
    )(x, icl)

    it3 = idx_token.reshape(_B, 1, _N)
    aw3 = agg_weight.astype(jnp.float32).reshape(_B, 1, _N)
    itn, awn = pl.pallas_call(
        _gather_kernel,
        grid=(_B, _RT),
        in_specs=[
            pl.BlockSpec((1, 1, _TM), lambda b, t: (b, 0, t)),
            pl.BlockSpec((1, 1, _N), lambda b, t: (b, 0, 0)),
            pl.BlockSpec((1, 1, _N), lambda b, t: (b, 0, 0)),
            pl.BlockSpec((1, 1, _TM), lambda b, t: (b, 0, t)),
        ],
        out_specs=[
            pl.BlockSpec((1, 1, _TM), lambda b, t: (b, 0, t)),
            pl.BlockSpec((1, 1, _TM), lambda b, t: (b, 0, t)),
        ],
        out_shape=[
            jax.ShapeDtypeStruct((_B, 1, _N), jnp.int32),
            jax.ShapeDtypeStruct((_B, 1, _N), jnp.float32),
        ],
    )(it3, icl, nw, aw3)

    return (xm, itn.reshape(_B, _N), awn.reshape(_B, _N, 1),
            icl.reshape(_B, _N), idown.reshape(_B, _CN))


# speedup vs baseline: 1.4498x; 1.2835x over previous
import jax
import jax.numpy as jnp
from jax.experimental import pallas as pl

_B, _N, _C = 4, 2048, 64
_K = 5
_CN = 512
_TM = 1024
_RT = _N // _TM
_SQRT_C = 8.0


def _cmp(a, b):
    if a is None:
        return (b, None)
    if b is None:
        return (a, None)
    return (jnp.minimum(a, b), jnp.maximum(a, b))


def _oemerge_rec(a, b):
    n = len(a)
    if n == 1:
        return list(_cmp(a[0], b[0]))
    e = _oemerge_rec(a[0::2], b[0::2])
    o = _oemerge_rec(a[1::2], b[1::2])
    out = [e[0]]
    for i in range(n - 1):
        lo, hi = _cmp(o[i], e[i + 1])
        out += [lo, hi]
    out.append(o[n - 1])
    return out


def _oemerge(a, b):
    n = max(len(a), len(b))
    n = 1 << (n - 1).bit_length()
    a = a + [None] * (n - len(a))
    b = b + [None] * (n - len(b))
    return _oemerge_rec(a, b)


def _bottom5(chunks):
    lists = [[c] for c in chunks]
    while len(lists) > 1:
        nxt = []
        for i in range(0, len(lists), 2):
            nxt.append(_oemerge(lists[i], lists[i + 1])[:_K])
        lists = nxt
    return lists[0][:_K]


def _dist_stats_kernel(xr_ref, xa_ref, sqr_ref, sqa_ref, dist_ref, dn_ref, dmax_ref):
    xr = xr_ref[0]
    xa = xa_ref[0]
    sqr = sqr_ref[0, 0][:, None]
    sqa = sqa_ref[0, 0][None, :]
    prod = jax.lax.dot_general(xr, xa, (((1,), (1,)), ((), ())),
                               preferred_element_type=jnp.float32)
    d2 = sqr + sqa - 2.0 * prod
    dist = jnp.sqrt(jnp.maximum(d2, 0.0)) / _SQRT_C
    dist_ref[0] = dist
    dmax_ref[0, 0] = jnp.max(dist, axis=1)

    chunks = [dist[:, i * 128:(i + 1) * 128] for i in range(_N // 128)]
    cand = jnp.concatenate(_bottom5(chunks), axis=1)
    ncand = cand.shape[1]
    col = jax.lax.broadcasted_iota(jnp.int32, (_TM, ncand), 1)
    cur = cand
    for r in range(_K):
        m = jnp.min(cur, axis=1, keepdims=True)
        dn_ref[0, r] = m[:, 0]
        if r < _K - 1:
            first = jnp.min(jnp.where(cur == m, col, ncand), axis=1, keepdims=True)
            cur = jnp.where(col == first, jnp.inf, cur)


def _score_kernel(dist_ref, densr_ref, densa_ref, dmax_ref, score_ref):
    dist = dist_ref[0]
    di = densr_ref[0, 0]
    da = densa_ref[0, 0][None, :]
    dm = dmax_ref[0, 0][:, None]
    masked = jnp.where(da > di[:, None], dist, dm)
    score_ref[0, 0] = jnp.min(masked, axis=1) * di


def _rank_kernel(sa_ref, sr_ref, rank_ref):
    sa = sa_ref[0, 0][None, :]
    si = sr_ref[0, 0][:, None]
    colj = jax.lax.broadcasted_iota(jnp.int32, (_TM, _N), 1)
    rowi = jax.lax.broadcasted_iota(jnp.int32, (_TM, _N), 0) + pl.program_id(1) * _TM
    gt = (sa > si) | ((sa == si) & (colj < rowi))
    rank_ref[0, 0] = jnp.sum(gt.astype(jnp.int32), axis=1)


def _centers_kernel(rank_ref, x_ref, sq_ref, idown_ref, xc_ref, sqc_ref):
    rank = rank_ref[0, 0][None, :]
    r_iota = jax.lax.broadcasted_iota(jnp.int32, (_CN, _N), 0)
    E = (rank == r_iota).astype(jnp.float32)
    i_iota = jax.lax.broadcasted_iota(jnp.int32, (_CN, _N), 1).astype(jnp.float32)
    idown_ref[0, 0] = jnp.sum(E * i_iota, axis=1).astype(jnp.int32)
    xc_ref[0] = jax.lax.dot_general(E, x_ref[0], (((1,), (0,)), ((), ())),
                                    preferred_element_type=jnp.float32)
    sqc_ref[0, 0] = jnp.sum(E * sq_ref[0, 0][None, :], axis=1)


def _assign_kernel(xr_ref, sqr_ref, xc_ref, sqc_ref, rank_ref, ic_ref):
    xr = xr_ref[0]
    sqr = sqr_ref[0, 0][None, :]
    xc = xc_ref[0]
    sqc = sqc_ref[0, 0][:, None]
    prod = jax.lax.dot_general(xc, xr, (((1,), (1,)), ((), ())),
                               preferred_element_type=jnp.float32)
    d2 = sqc + sqr - 2.0 * prod
    distc = jnp.sqrt(jnp.maximum(d2, 0.0)) / _SQRT_C
    minv = jnp.min(distc, axis=0, keepdims=True)
    kio = jax.lax.broadcasted_iota(jnp.int32, (_CN, _TM), 0)
    ic = jnp.min(jnp.where(distc == minv, kio, _CN), axis=0)
    rank = rank_ref[0, 0]
    ic_ref[0, 0] = jnp.where(rank < _CN, rank, ic)


def _merge_kernel(x_ref, ic_ref, xm_ref, nw_ref):
    ic = ic_ref[0, 0]
    kio = jax.lax.broadcasted_iota(jnp.int32, (_CN, _N), 0)
    A = (ic[None, :] == kio).astype(jnp.float32)
    count = jnp.sum(A, axis=1)
    inv = 1.0 / (count + 1e-06)
    nw = jnp.sum(A * inv[:, None], axis=0)
    nw_ref[0, 0] = nw
    xw = x_ref[0] * nw[:, None]
    xm_ref[0] = jax.lax.dot_general(A, xw, (((1,), (0,)), ((), ())),
                                    preferred_element_type=jnp.float32)


def _gather_kernel(it_ref, ic_ref, nw_ref, aw_ref, itn_ref, awn_ref):
    it = it_ref[0, 0]
    icf = ic_ref[0, 0].astype(jnp.float32)[None, :]
    nw = nw_ref[0, 0][None, :]
    mio = jax.lax.broadcasted_iota(jnp.int32, (_TM, _N), 1)
    G = it[:, None] == mio
    itn_ref[0, 0] = jnp.sum(jnp.where(G, icf, 0.0), axis=1).astype(jnp.int32)
    wt = jnp.sum(jnp.where(G, nw, 0.0), axis=1)
    awn_ref[0, 0] = aw_ref[0, 0] * wt


def kernel(x, idx_token, agg_token, agg_weight):
    if agg_weight is None:
        agg_weight = agg_token
    x = x.astype(jnp.float32)
    sq = jnp.sum(x * x, axis=-1)
    sq3 = sq.reshape(_B, 1, _N)

    dist, dn, dmax = pl.pallas_call(
        _dist_stats_kernel,
        grid=(_B, _RT),
        in_specs=[
            pl.BlockSpec((1, _TM, _C), lambda b, t: (b, t, 0)),
            pl.BlockSpec((1, _N, _C), lambda b, t: (b, 0, 0)),
            pl.BlockSpec((1, 1, _TM), lambda b, t: (b, 0, t)),
            pl.BlockSpec((1, 1, _N), lambda b, t: (b, 0, 0)),
        ],
        out_specs=[
            pl.BlockSpec((1, _TM, _N), lambda b, t: (b, t, 0)),
            pl.BlockSpec((1, _K, _TM), lambda b, t: (b, 0, t)),
            pl.BlockSpec((1, 1, _TM), lambda b, t: (b, 0, t)),
        ],
        out_shape=[
            jax.ShapeDtypeStruct((_B, _N, _N), jnp.float32),
            jax.ShapeDtypeStruct((_B, _K, _N), jnp.float32),
            jax.ShapeDtypeStruct((_B, 1, _N), jnp.float32),
        ],
    )(x, x, sq3, sq3)

    dn_t = jnp.transpose(dn, (0, 2, 1))
    dens_flat = jnp.exp(-(dn_t ** 2).mean(axis=-1))
    dens_flat = dens_flat + jax.random.uniform(
        jax.random.key(1), dens_flat.shape, dtype=dens_flat.dtype) * 1e-06
    dens = dens_flat.reshape(_B, 1, _N)

    score = pl.pallas_call(
        _score_kernel,
        grid=(_B, _RT),
        in_specs=[
            pl.BlockSpec((1, _TM, _N), lambda b, t: (b, t, 0)),
            pl.BlockSpec((1, 1, _TM), lambda b, t: (b, 0, t)),
            pl.BlockSpec((1, 1, _N), lambda b, t: (b, 0, 0)),
            pl.BlockSpec((1, 1, _TM), lambda b, t: (b, 0, t)),
        ],
        out_specs=pl.BlockSpec((1, 1, _TM), lambda b, t: (b, 0, t)),
        out_shape=jax.ShapeDtypeStruct((_B, 1, _N), jnp.float32),
    )(dist, dens, dens, dmax)

    rank = pl.pallas_call(
        _rank_kernel,
        grid=(_B, _RT),
        in_specs=[
            pl.BlockSpec((1, 1, _N), lambda b, t: (b, 0, 0)),
            pl.BlockSpec((1, 1, _TM), lambda b, t: (b, 0, t)),
        ],
        out_specs=pl.BlockSpec((1, 1, _TM), lambda b, t: (b, 0, t)),
        out_shape=jax.ShapeDtypeStruct((_B, 1, _N), jnp.int32),
    )(score, score)

    idown, xc, sqc = pl.pallas_call(
        _centers_kernel,
        grid=(_B,),
        in_specs=[
            pl.BlockSpec((1, 1, _N), lambda b: (b, 0, 0)),
            pl.BlockSpec((1, _N, _C), lambda b: (b, 0, 0)),
            pl.BlockSpec((1, 1, _N), lambda b: (b, 0, 0)),
        ],
        out_specs=[
            pl.BlockSpec((1, 1, _CN), lambda b: (b, 0, 0)),
            pl.BlockSpec((1, _CN, _C), lambda b: (b, 0, 0)),
            pl.BlockSpec((1, 1, _CN), lambda b: (b, 0, 0)),
        ],
        out_shape=[
            jax.ShapeDtypeStruct((_B, 1, _CN), jnp.int32),
            jax.ShapeDtypeStruct((_B, _CN, _C), jnp.float32),
            jax.ShapeDtypeStruct((_B, 1, _CN), jnp.float32),
        ],
    )(rank, x, sq3)

    icl = pl.pallas_call(
        _assign_kernel,
        grid=(_B, _RT),
        in_specs=[
            pl.BlockSpec((1, _TM, _C), lambda b, t: (b, t, 0)),
            pl.BlockSpec((1, 1, _TM), lambda b, t: (b, 0, t)),
            pl.BlockSpec((1, _CN, _C), lambda b, t: (b, 0, 0)),
            pl.BlockSpec((1, 1, _CN), lambda b, t: (b, 0, 0)),
            pl.BlockSpec((1, 1, _TM), lambda b, t: (b, 0, t)),
        ],
        out_specs=pl.BlockSpec((1, 1, _TM), lambda b, t: (b, 0, t)),
        out_shape=jax.ShapeDtypeStruct((_B, 1, _N), jnp.int32),
    )(x, sq3, xc, sqc, rank)

    xm, nw = pl.pallas_call(
        _merge_kernel,
        grid=(_B,),
        in_specs=[
            pl.BlockSpec((1, _N, _C), lambda b: (b, 0, 0)),
            pl.BlockSpec((1, 1, _N), lambda b: (b, 0, 0)),
        ],
        out_specs=[
            pl.BlockSpec((1, _CN, _C), lambda b: (b, 0, 0)),
            pl.BlockSpec((1, 1, _N), lambda b: (b, 0, 0)),
        ],
        out_shape=[
            jax.ShapeDtypeStruct((_B, _CN, _C), jnp.float32),
            jax.ShapeDtypeStruct((_B, 1, _N), jnp.float32),
        ],
    )(x, icl)

    it3 = idx_token.reshape(_B, 1, _N)
    aw3 = agg_weight.astype(jnp.float32).reshape(_B, 1, _N)
    itn, awn = pl.pallas_call(
        _gather_kernel,
        grid=(_B, _RT),
        in_specs=[
            pl.BlockSpec((1, 1, _TM), lambda b, t: (b, 0, t)),
            pl.BlockSpec((1, 1, _N), lambda b, t: (b, 0, 0)),
            pl.BlockSpec((1, 1, _N), lambda b, t: (b, 0, 0)),
            pl.BlockSpec((1, 1, _TM), lambda b, t: (b, 0, t)),
        ],
        out_specs=[
            pl.BlockSpec((1, 1, _TM), lambda b, t: (b, 0, t)),
            pl.BlockSpec((1, 1, _TM), lambda b, t: (b, 0, t)),
        ],
        out_shape=[
            jax.ShapeDtypeStruct((_B, 1, _N), jnp.int32),
            jax.ShapeDtypeStruct((_B, 1, _N), jnp.float32),
        ],
    )(it3, icl, nw, aw3)

    return (xm, itn.reshape(_B, _N), awn.reshape(_B, _N, 1),
            icl.reshape(_B, _N), idown.reshape(_B, _CN))


# speedup vs baseline: 1.4924x; 1.0294x over previous
import jax
import jax.numpy as jnp
from jax.experimental import pallas as pl

_B, _N, _C = 4, 2048, 64
_K = 5
_CN = 512
_TM = 2048
_RT = _N // _TM
_SQRT_C = 8.0


def _cmp(a, b):
    if a is None:
        return (b, None)
    if b is None:
        return (a, None)
    return (jnp.minimum(a, b), jnp.maximum(a, b))


def _oemerge_rec(a, b):
    n = len(a)
    if n == 1:
        return list(_cmp(a[0], b[0]))
    e = _oemerge_rec(a[0::2], b[0::2])
    o = _oemerge_rec(a[1::2], b[1::2])
    out = [e[0]]
    for i in range(n - 1):
        lo, hi = _cmp(o[i], e[i + 1])
        out += [lo, hi]
    out.append(o[n - 1])
    return out


def _oemerge(a, b):
    n = max(len(a), len(b))
    n = 1 << (n - 1).bit_length()
    a = a + [None] * (n - len(a))
    b = b + [None] * (n - len(b))
    return _oemerge_rec(a, b)


def _bottom5(chunks):
    lists = [[c] for c in chunks]
    while len(lists) > 1:
        nxt = []
        for i in range(0, len(lists), 2):
            nxt.append(_oemerge(lists[i], lists[i + 1])[:_K])
        lists = nxt
    return lists[0][:_K]


def _dist_stats_kernel(xr_ref, xa_ref, sqr_ref, sqa_ref, dist_ref, dn_ref, dmax_ref):
    xr = xr_ref[0]
    xa = xa_ref[0]
    sqr = sqr_ref[0, 0][:, None]
    sqa = sqa_ref[0, 0][None, :]
    prod = jax.lax.dot_general(xr, xa, (((1,), (1,)), ((), ())),
                               preferred_element_type=jnp.float32)
    d2 = sqr + sqa - 2.0 * prod
    dist = jnp.sqrt(jnp.maximum(d2, 0.0)) / _SQRT_C
    dist_ref[0] = dist
    dmax_ref[0, 0] = jnp.max(dist, axis=1)

    chunks = [dist[:, i * 128:(i + 1) * 128] for i in range(_N // 128)]
    cand = jnp.concatenate(_bottom5(chunks), axis=1)
    ncand = cand.shape[1]
    col = jax.lax.broadcasted_iota(jnp.int32, (_TM, ncand), 1)
    cur = cand
    for r in range(_K):
        m = jnp.min(cur, axis=1, keepdims=True)
        dn_ref[0, r] = m[:, 0]
        if r < _K - 1:
            first = jnp.min(jnp.where(cur == m, col, ncand), axis=1, keepdims=True)
            cur = jnp.where(col == first, jnp.inf, cur)


def _score_kernel(dist_ref, densr_ref, densa_ref, dmax_ref, score_ref):
    dist = dist_ref[0]
    di = densr_ref[0, 0]
    da = densa_ref[0, 0][None, :]
    dm = dmax_ref[0, 0][:, None]
    masked = jnp.where(da > di[:, None], dist, dm)
    score_ref[0, 0] = jnp.min(masked, axis=1) * di


def _rank_kernel(sa_ref, sr_ref, rank_ref):
    sa = sa_ref[0, 0][None, :]
    si = sr_ref[0, 0][:, None]
    colj = jax.lax.broadcasted_iota(jnp.int32, (_TM, _N), 1)
    rowi = jax.lax.broadcasted_iota(jnp.int32, (_TM, _N), 0) + pl.program_id(1) * _TM
    gt = (sa > si) | ((sa == si) & (colj < rowi))
    rank_ref[0, 0] = jnp.sum(gt.astype(jnp.int32), axis=1)


def _centers_kernel(rank_ref, x_ref, sq_ref, idown_ref, xc_ref, sqc_ref):
    rank = rank_ref[0, 0][None, :]
    r_iota = jax.lax.broadcasted_iota(jnp.int32, (_CN, _N), 0)
    E = (rank == r_iota).astype(jnp.float32)
    i_iota = jax.lax.broadcasted_iota(jnp.int32, (_CN, _N), 1).astype(jnp.float32)
    idown_ref[0, 0] = jnp.sum(E * i_iota, axis=1).astype(jnp.int32)
    xc_ref[0] = jax.lax.dot_general(E, x_ref[0], (((1,), (0,)), ((), ())),
                                    preferred_element_type=jnp.float32)
    sqc_ref[0, 0] = jnp.sum(E * sq_ref[0, 0][None, :], axis=1)


def _assign_kernel(xr_ref, sqr_ref, xc_ref, sqc_ref, rank_ref, ic_ref):
    xr = xr_ref[0]
    sqr = sqr_ref[0, 0][None, :]
    xc = xc_ref[0]
    sqc = sqc_ref[0, 0][:, None]
    prod = jax.lax.dot_general(xc, xr, (((1,), (1,)), ((), ())),
                               preferred_element_type=jnp.float32)
    d2 = sqc + sqr - 2.0 * prod
    distc = jnp.sqrt(jnp.maximum(d2, 0.0)) / _SQRT_C
    minv = jnp.min(distc, axis=0, keepdims=True)
    kio = jax.lax.broadcasted_iota(jnp.int32, (_CN, _TM), 0)
    ic = jnp.min(jnp.where(distc == minv, kio, _CN), axis=0)
    rank = rank_ref[0, 0]
    ic_ref[0, 0] = jnp.where(rank < _CN, rank, ic)


def _merge_kernel(x_ref, ic_ref, xm_ref, nw_ref):
    ic = ic_ref[0, 0]
    kio = jax.lax.broadcasted_iota(jnp.int32, (_CN, _N), 0)
    A = (ic[None, :] == kio).astype(jnp.float32)
    count = jnp.sum(A, axis=1)
    inv = 1.0 / (count + 1e-06)
    nw = jnp.sum(A * inv[:, None], axis=0)
    nw_ref[0, 0] = nw
    xw = x_ref[0] * nw[:, None]
    xm_ref[0] = jax.lax.dot_general(A, xw, (((1,), (0,)), ((), ())),
                                    preferred_element_type=jnp.float32)


def _gather_kernel(it_ref, ic_ref, nw_ref, aw_ref, itn_ref, awn_ref):
    it = it_ref[0, 0]
    icf = ic_ref[0, 0].astype(jnp.float32)[None, :]
    nw = nw_ref[0, 0][None, :]
    mio = jax.lax.broadcasted_iota(jnp.int32, (_TM, _N), 1)
    G = it[:, None] == mio
    itn_ref[0, 0] = jnp.sum(jnp.where(G, icf, 0.0), axis=1).astype(jnp.int32)
    wt = jnp.sum(jnp.where(G, nw, 0.0), axis=1)
    awn_ref[0, 0] = aw_ref[0, 0] * wt


def kernel(x, idx_token, agg_token, agg_weight):
    if agg_weight is None:
        agg_weight = agg_token
    x = x.astype(jnp.float32)
    sq = jnp.sum(x * x, axis=-1)
    sq3 = sq.reshape(_B, 1, _N)

    dist, dn, dmax = pl.pallas_call(
        _dist_stats_kernel,
        grid=(_B, _RT),
        in_specs=[
            pl.BlockSpec((1, _TM, _C), lambda b, t: (b, t, 0)),
            pl.BlockSpec((1, _N, _C), lambda b, t: (b, 0, 0)),
            pl.BlockSpec((1, 1, _TM), lambda b, t: (b, 0, t)),
            pl.BlockSpec((1, 1, _N), lambda b, t: (b, 0, 0)),
        ],
        out_specs=[
            pl.BlockSpec((1, _TM, _N), lambda b, t: (b, t, 0)),
            pl.BlockSpec((1, _K, _TM), lambda b, t: (b, 0, t)),
            pl.BlockSpec((1, 1, _TM), lambda b, t: (b, 0, t)),
        ],
        out_shape=[
            jax.ShapeDtypeStruct((_B, _N, _N), jnp.float32),
            jax.ShapeDtypeStruct((_B, _K, _N), jnp.float32),
            jax.ShapeDtypeStruct((_B, 1, _N), jnp.float32),
        ],
    )(x, x, sq3, sq3)

    dn_t = jnp.transpose(dn, (0, 2, 1))
    dens_flat = jnp.exp(-(dn_t ** 2).mean(axis=-1))
    dens_flat = dens_flat + jax.random.uniform(
        jax.random.key(1), dens_flat.shape, dtype=dens_flat.dtype) * 1e-06
    dens = dens_flat.reshape(_B, 1, _N)

    score = pl.pallas_call(
        _score_kernel,
        grid=(_B, _RT),
        in_specs=[
            pl.BlockSpec((1, _TM, _N), lambda b, t: (b, t, 0)),
            pl.BlockSpec((1, 1, _TM), lambda b, t: (b, 0, t)),
            pl.BlockSpec((1, 1, _N), lambda b, t: (b, 0, 0)),
            pl.BlockSpec((1, 1, _TM), lambda b, t: (b, 0, t)),
        ],
        out_specs=pl.BlockSpec((1, 1, _TM), lambda b, t: (b, 0, t)),
        out_shape=jax.ShapeDtypeStruct((_B, 1, _N), jnp.float32),
    )(dist, dens, dens, dmax)

    rank = pl.pallas_call(
        _rank_kernel,
        grid=(_B, _RT),
        in_specs=[
            pl.BlockSpec((1, 1, _N), lambda b, t: (b, 0, 0)),
            pl.BlockSpec((1, 1, _TM), lambda b, t: (b, 0, t)),
        ],
        out_specs=pl.BlockSpec((1, 1, _TM), lambda b, t: (b, 0, t)),
        out_shape=jax.ShapeDtypeStruct((_B, 1, _N), jnp.int32),
    )(score, score)

    idown, xc, sqc = pl.pallas_call(
        _centers_kernel,
        grid=(_B,),
        in_specs=[
            pl.BlockSpec((1, 1, _N), lambda b: (b, 0, 0)),
            pl.BlockSpec((1, _N, _C), lambda b: (b, 0, 0)),
            pl.BlockSpec((1, 1, _N), lambda b: (b, 0, 0)),
        ],
        out_specs=[
            pl.BlockSpec((1, 1, _CN), lambda b: (b, 0, 0)),
            pl.BlockSpec((1, _CN, _C), lambda b: (b, 0, 0)),
            pl.BlockSpec((1, 1, _CN), lambda b: (b, 0, 0)),
        ],
        out_shape=[
            jax.ShapeDtypeStruct((_B, 1, _CN), jnp.int32),
            jax.ShapeDtypeStruct((_B, _CN, _C), jnp.float32),
            jax.ShapeDtypeStruct((_B, 1, _CN), jnp.float32),
        ],
    )(rank, x, sq3)

    icl = pl.pallas_call(
        _assign_kernel,
        grid=(_B, _RT),
        in_specs=[
            pl.BlockSpec((1, _TM, _C), lambda b, t: (b, t, 0)),
            pl.BlockSpec((1, 1, _TM), lambda b, t: (b, 0, t)),
            pl.BlockSpec((1, _CN, _C), lambda b, t: (b, 0, 0)),
            pl.BlockSpec((1, 1, _CN), lambda b, t: (b, 0, 0)),
            pl.BlockSpec((1, 1, _TM), lambda b, t: (b, 0, t)),
        ],
        out_specs=pl.BlockSpec((1, 1, _TM), lambda b, t: (b, 0, t)),
        out_shape=jax.ShapeDtypeStruct((_B, 1, _N), jnp.int32),
    )(x, sq3, xc, sqc, rank)

    xm, nw = pl.pallas_call(
        _merge_kernel,
        grid=(_B,),
        in_specs=[
            pl.BlockSpec((1, _N, _C), lambda b: (b, 0, 0)),
            pl.BlockSpec((1, 1, _N), lambda b: (b, 0, 0)),
        ],
        out_specs=[
            pl.BlockSpec((1, _CN, _C), lambda b: (b, 0, 0)),
            pl.BlockSpec((1, 1, _N), lambda b: (b, 0, 0)),
        ],
        out_shape=[
            jax.ShapeDtypeStruct((_B, _CN, _C), jnp.float32),
            jax.ShapeDtypeStruct((_B, 1, _N), jnp.float32),
        ],
    )(x, icl)

    it3 = idx_token.reshape(_B, 1, _N)
    aw3 = agg_weight.astype(jnp.float32).reshape(_B, 1, _N)
    itn, awn = pl.pallas_call(
        _gather_kernel,
        grid=(_B, _RT),
        in_specs=[
            pl.BlockSpec((1, 1, _TM), lambda b, t: (b, 0, t)),
            pl.BlockSpec((1, 1, _N), lambda b, t: (b, 0, 0)),
            pl.BlockSpec((1, 1, _N), lambda b, t: (b, 0, 0)),
            pl.BlockSpec((1, 1, _TM), lambda b, t: (b, 0, t)),
        ],
        out_specs=[
            pl.BlockSpec((1, 1, _TM), lambda b, t: (b, 0, t)),
            pl.BlockSpec((1, 1, _TM), lambda b, t: (b, 0, t)),
        ],
        out_shape=[
            jax.ShapeDtypeStruct((_B, 1, _N), jnp.int32),
            jax.ShapeDtypeStruct((_B, 1, _N), jnp.float32),
        ],
    )(it3, icl, nw, aw3)

    return (xm, itn.reshape(_B, _N), awn.reshape(_B, _N, 1),
            icl.reshape(_B, _N), idown.reshape(_B, _CN))


# speedup vs baseline: 1.5224x; 1.0201x over previous
import jax
import jax.numpy as jnp
from jax.experimental import pallas as pl

_B, _N, _C = 4, 2048, 64
_K = 5
_CN = 512
_TM = 2048
_RT = _N // _TM
_SQRT_C = 8.0


def _cmp(a, b):
    if a is None:
        return (b, None)
    if b is None:
        return (a, None)
    return (jnp.minimum(a, b), jnp.maximum(a, b))


def _oemerge_rec(a, b):
    n = len(a)
    if n == 1:
        return list(_cmp(a[0], b[0]))
    e = _oemerge_rec(a[0::2], b[0::2])
    o = _oemerge_rec(a[1::2], b[1::2])
    out = [e[0]]
    for i in range(n - 1):
        lo, hi = _cmp(o[i], e[i + 1])
        out += [lo, hi]
    out.append(o[n - 1])
    return out


def _oemerge(a, b):
    n = max(len(a), len(b))
    n = 1 << (n - 1).bit_length()
    a = a + [None] * (n - len(a))
    b = b + [None] * (n - len(b))
    return _oemerge_rec(a, b)


def _bottom5(chunks):
    lists = [[c] for c in chunks]
    while len(lists) > 1:
        nxt = []
        for i in range(0, len(lists), 2):
            nxt.append(_oemerge(lists[i], lists[i + 1])[:_K])
        lists = nxt
    return lists[0][:_K]


def _dist_stats_kernel(xr_ref, xa_ref, sqr_ref, sqa_ref, dist_ref, dn_ref, dmax_ref):
    xr = xr_ref[0]
    xa = xa_ref[0]
    sqr = sqr_ref[0, 0][:, None]
    sqa = sqa_ref[0, 0][None, :]
    prod = jax.lax.dot_general(xr, xa, (((1,), (1,)), ((), ())),
                               preferred_element_type=jnp.float32)
    d2 = sqr + sqa - 2.0 * prod
    dist = jnp.sqrt(jnp.maximum(d2, 0.0)) / _SQRT_C
    dist_ref[0] = dist
    dmax_ref[0, 0] = jnp.max(dist, axis=1)

    chunks = [dist[:, i * 128:(i + 1) * 128] for i in range(_N // 128)]
    cand = jnp.concatenate(_bottom5(chunks), axis=1)
    ncand = cand.shape[1]
    col = jax.lax.broadcasted_iota(jnp.int32, (_TM, ncand), 1)
    cur = cand
    for r in range(_K):
        m = jnp.min(cur, axis=1, keepdims=True)
        dn_ref[0, r] = m[:, 0]
        if r < _K - 1:
            first = jnp.min(jnp.where(cur == m, col, ncand), axis=1, keepdims=True)
            cur = jnp.where(col == first, jnp.inf, cur)


def _score_kernel(dist_ref, densr_ref, densa_ref, dmax_ref, score_ref):
    dist = dist_ref[0]
    di = densr_ref[0, 0]
    da = densa_ref[0, 0][None, :]
    dm = dmax_ref[0, 0][:, None]
    masked = jnp.where(da > di[:, None], dist, dm)
    score_ref[0, 0] = jnp.min(masked, axis=1) * di


def _rank_kernel(sa_ref, sr_ref, rank_ref):
    sa = sa_ref[0, 0][None, :]
    si = sr_ref[0, 0][:, None]
    colj = jax.lax.broadcasted_iota(jnp.int32, (_TM, _N), 1)
    rowi = jax.lax.broadcasted_iota(jnp.int32, (_TM, _N), 0) + pl.program_id(1) * _TM
    gt = (sa > si) | ((sa == si) & (colj < rowi))
    rank_ref[0, 0] = jnp.sum(gt.astype(jnp.int32), axis=1)


def _centers_assign_kernel(rank_ref, x_ref, sq_ref, idown_ref, ic_ref):
    rank = rank_ref[0, 0]
    x = x_ref[0]
    sq = sq_ref[0, 0]
    r_iota = jax.lax.broadcasted_iota(jnp.int32, (_CN, _N), 0)
    E = (rank[None, :] == r_iota).astype(jnp.float32)
    i_iota = jax.lax.broadcasted_iota(jnp.int32, (_CN, _N), 1).astype(jnp.float32)
    idown_ref[0, 0] = jnp.sum(E * i_iota, axis=1).astype(jnp.int32)
    xc = jax.lax.dot_general(E, x, (((1,), (0,)), ((), ())),
                             preferred_element_type=jnp.float32)
    sqc = jnp.sum(E * sq[None, :], axis=1)
    prod = jax.lax.dot_general(xc, x, (((1,), (1,)), ((), ())),
                               preferred_element_type=jnp.float32)
    d2 = sqc[:, None] + sq[None, :] - 2.0 * prod
    distc = jnp.sqrt(jnp.maximum(d2, 0.0)) / _SQRT_C
    minv = jnp.min(distc, axis=0, keepdims=True)
    kio = jax.lax.broadcasted_iota(jnp.int32, (_CN, _N), 0)
    ic = jnp.min(jnp.where(distc == minv, kio, _CN), axis=0)
    ic_ref[0, 0] = jnp.where(rank < _CN, rank, ic)


def _merge_kernel(x_ref, ic_ref, xm_ref, nw_ref):
    ic = ic_ref[0, 0]
    kio = jax.lax.broadcasted_iota(jnp.int32, (_CN, _N), 0)
    A = (ic[None, :] == kio).astype(jnp.float32)
    count = jnp.sum(A, axis=1)
    inv = 1.0 / (count + 1e-06)
    nw = jnp.sum(A * inv[:, None], axis=0)
    nw_ref[0, 0] = nw
    xw = x_ref[0] * nw[:, None]
    xm_ref[0] = jax.lax.dot_general(A, xw, (((1,), (0,)), ((), ())),
                                    preferred_element_type=jnp.float32)


def _gather_kernel(it_ref, ic_ref, nw_ref, aw_ref, itn_ref, awn_ref):
    it = it_ref[0, 0]
    icf = ic_ref[0, 0].astype(jnp.float32)[None, :]
    nw = nw_ref[0, 0][None, :]
    mio = jax.lax.broadcasted_iota(jnp.int32, (_TM, _N), 1)
    G = it[:, None] == mio
    itn_ref[0, 0] = jnp.sum(jnp.where(G, icf, 0.0), axis=1).astype(jnp.int32)
    wt = jnp.sum(jnp.where(G, nw, 0.0), axis=1)
    awn_ref[0, 0] = aw_ref[0, 0] * wt


def kernel(x, idx_token, agg_token, agg_weight):
    if agg_weight is None:
        agg_weight = agg_token
    x = x.astype(jnp.float32)
    sq = jnp.sum(x * x, axis=-1)
    sq3 = sq.reshape(_B, 1, _N)

    dist, dn, dmax = pl.pallas_call(
        _dist_stats_kernel,
        grid=(_B, _RT),
        in_specs=[
            pl.BlockSpec((1, _TM, _C), lambda b, t: (b, t, 0)),
            pl.BlockSpec((1, _N, _C), lambda b, t: (b, 0, 0)),
            pl.BlockSpec((1, 1, _TM), lambda b, t: (b, 0, t)),
            pl.BlockSpec((1, 1, _N), lambda b, t: (b, 0, 0)),
        ],
        out_specs=[
            pl.BlockSpec((1, _TM, _N), lambda b, t: (b, t, 0)),
            pl.BlockSpec((1, _K, _TM), lambda b, t: (b, 0, t)),
            pl.BlockSpec((1, 1, _TM), lambda b, t: (b, 0, t)),
        ],
        out_shape=[
            jax.ShapeDtypeStruct((_B, _N, _N), jnp.float32),
            jax.ShapeDtypeStruct((_B, _K, _N), jnp.float32),
            jax.ShapeDtypeStruct((_B, 1, _N), jnp.float32),
        ],
    )(x, x, sq3, sq3)

    dn_t = jnp.transpose(dn, (0, 2, 1))
    dens_flat = jnp.exp(-(dn_t ** 2).mean(axis=-1))
    dens_flat = dens_flat + jax.random.uniform(
        jax.random.key(1), dens_flat.shape, dtype=dens_flat.dtype) * 1e-06
    dens = dens_flat.reshape(_B, 1, _N)

    score = pl.pallas_call(
        _score_kernel,
        grid=(_B, _RT),
        in_specs=[
            pl.BlockSpec((1, _TM, _N), lambda b, t: (b, t, 0)),
            pl.BlockSpec((1, 1, _TM), lambda b, t: (b, 0, t)),
            pl.BlockSpec((1, 1, _N), lambda b, t: (b, 0, 0)),
            pl.BlockSpec((1, 1, _TM), lambda b, t: (b, 0, t)),
        ],
        out_specs=pl.BlockSpec((1, 1, _TM), lambda b, t: (b, 0, t)),
        out_shape=jax.ShapeDtypeStruct((_B, 1, _N), jnp.float32),
    )(dist, dens, dens, dmax)

    rank = pl.pallas_call(
        _rank_kernel,
        grid=(_B, _RT),
        in_specs=[
            pl.BlockSpec((1, 1, _N), lambda b, t: (b, 0, 0)),
            pl.BlockSpec((1, 1, _TM), lambda b, t: (b, 0, t)),
        ],
        out_specs=pl.BlockSpec((1, 1, _TM), lambda b, t: (b, 0, t)),
        out_shape=jax.ShapeDtypeStruct((_B, 1, _N), jnp.int32),
    )(score, score)

    idown, icl = pl.pallas_call(
        _centers_assign_kernel,
        grid=(_B,),
        in_specs=[
            pl.BlockSpec((1, 1, _N), lambda b: (b, 0, 0)),
            pl.BlockSpec((1, _N, _C), lambda b: (b, 0, 0)),
            pl.BlockSpec((1, 1, _N), lambda b: (b, 0, 0)),
        ],
        out_specs=[
            pl.BlockSpec((1, 1, _CN), lambda b: (b, 0, 0)),
            pl.BlockSpec((1, 1, _N), lambda b: (b, 0, 0)),
        ],
        out_shape=[
            jax.ShapeDtypeStruct((_B, 1, _CN), jnp.int32),
            jax.ShapeDtypeStruct((_B, 1, _N), jnp.int32),
        ],
    )(rank, x, sq3)

    xm, nw = pl.pallas_call(
        _merge_kernel,
        grid=(_B,),
        in_specs=[
            pl.BlockSpec((1, _N, _C), lambda b: (b, 0, 0)),
            pl.BlockSpec((1, 1, _N), lambda b: (b, 0, 0)),
        ],
        out_specs=[
            pl.BlockSpec((1, _CN, _C), lambda b: (b, 0, 0)),
            pl.BlockSpec((1, 1, _N), lambda b: (b, 0, 0)),
        ],
        out_shape=[
            jax.ShapeDtypeStruct((_B, _CN, _C), jnp.float32),
            jax.ShapeDtypeStruct((_B, 1, _N), jnp.float32),
        ],
    )(x, icl)

    it3 = idx_token.reshape(_B, 1, _N)
    aw3 = agg_weight.astype(jnp.float32).reshape(_B, 1, _N)
    itn, awn = pl.pallas_call(
        _gather_kernel,
        grid=(_B, _RT),
        in_specs=[
            pl.BlockSpec((1, 1, _TM), lambda b, t: (b, 0, t)),
            pl.BlockSpec((1, 1, _N), lambda b, t: (b, 0, 0)),
            pl.BlockSpec((1, 1, _N), lambda b, t: (b, 0, 0)),
            pl.BlockSpec((1, 1, _TM), lambda b, t: (b, 0, t)),
        ],
        out_specs=[
            pl.BlockSpec((1, 1, _TM), lambda b, t: (b, 0, t)),
            pl.BlockSpec((1, 1, _TM), lambda b, t: (b, 0, t)),
        ],
        out_shape=[
            jax.ShapeDtypeStruct((_B, 1, _N), jnp.int32),
            jax.ShapeDtypeStruct((_B, 1, _N), jnp.float32),
        ],
    )(it3, icl, nw, aw3)

    return (xm, itn.reshape(_B, _N), awn.reshape(_B, _N, 1),
            icl.reshape(_B, _N), idown.reshape(_B, _CN))


# speedup vs baseline: 1.5442x; 1.0143x over previous
import jax
import jax.numpy as jnp
from jax.experimental import pallas as pl

_B, _N, _C = 4, 2048, 64
_K = 5
_CN = 512
_TM = 2048
_RT = _N // _TM
_SQRT_C = 8.0


def _cmp(a, b):
    if a is None:
        return (b, None)
    if b is None:
        return (a, None)
    return (jnp.minimum(a, b), jnp.maximum(a, b))


def _oemerge_rec(a, b):
    n = len(a)
    if n == 1:
        return list(_cmp(a[0], b[0]))
    e = _oemerge_rec(a[0::2], b[0::2])
    o = _oemerge_rec(a[1::2], b[1::2])
    out = [e[0]]
    for i in range(n - 1):
        lo, hi = _cmp(o[i], e[i + 1])
        out += [lo, hi]
    out.append(o[n - 1])
    return out


def _oemerge(a, b):
    n = max(len(a), len(b))
    n = 1 << (n - 1).bit_length()
    a = a + [None] * (n - len(a))
    b = b + [None] * (n - len(b))
    return _oemerge_rec(a, b)


def _bottom5(chunks):
    lists = [[c] for c in chunks]
    while len(lists) > 1:
        nxt = []
        for i in range(0, len(lists), 2):
            nxt.append(_oemerge(lists[i], lists[i + 1])[:_K])
        lists = nxt
    return lists[0][:_K]


def _dist_stats_kernel(xr_ref, xa_ref, sqr_ref, sqa_ref, dist_ref, dn_ref, dmax_ref):
    xr = xr_ref[0]
    xa = xa_ref[0]
    sqr = sqr_ref[0, 0][:, None]
    sqa = sqa_ref[0, 0][None, :]
    prod = jax.lax.dot_general(xr, xa, (((1,), (1,)), ((), ())),
                               preferred_element_type=jnp.float32)
    d2 = sqr + sqa - 2.0 * prod
    dist = jnp.sqrt(jnp.maximum(d2, 0.0)) / _SQRT_C
    dist_ref[0] = dist
    dmax_ref[0, 0] = jnp.max(dist, axis=1)

    chunks = [dist[:, i * 128:(i + 1) * 128] for i in range(_N // 128)]
    cand = jnp.concatenate(_bottom5(chunks), axis=1)
    ncand = cand.shape[1]
    col = jax.lax.broadcasted_iota(jnp.int32, (_TM, ncand), 1)
    cur = cand
    for r in range(_K):
        m = jnp.min(cur, axis=1, keepdims=True)
        dn_ref[0, r] = m[:, 0]
        if r < _K - 1:
            first = jnp.min(jnp.where(cur == m, col, ncand), axis=1, keepdims=True)
            cur = jnp.where(col == first, jnp.inf, cur)


def _score_kernel(dist_ref, densr_ref, densa_ref, dmax_ref, score_ref):
    dist = dist_ref[0]
    di = densr_ref[0, 0]
    da = densa_ref[0, 0][None, :]
    dm = dmax_ref[0, 0][:, None]
    masked = jnp.where(da > di[:, None], dist, dm)
    score_ref[0, 0] = jnp.min(masked, axis=1) * di


def _rank_kernel(sa_ref, sr_ref, rank_ref):
    sa = sa_ref[0, 0][None, :]
    si = sr_ref[0, 0][:, None]
    colj = jax.lax.broadcasted_iota(jnp.int32, (_TM, _N), 1)
    rowi = jax.lax.broadcasted_iota(jnp.int32, (_TM, _N), 0) + pl.program_id(1) * _TM
    gt = (sa > si) | ((sa == si) & (colj < rowi))
    rank_ref[0, 0] = jnp.sum(gt.astype(jnp.int32), axis=1)


def _centers_assign_kernel(rank_ref, x_ref, sq_ref, idown_ref, ic_ref):
    rank = rank_ref[0, 0]
    x = x_ref[0]
    sq = sq_ref[0, 0]
    r_iota = jax.lax.broadcasted_iota(jnp.int32, (_CN, _N), 0)
    E = (rank[None, :] == r_iota).astype(jnp.float32)
    i_iota = jax.lax.broadcasted_iota(jnp.int32, (_CN, _N), 1).astype(jnp.float32)
    idown_ref[0, 0] = jnp.sum(E * i_iota, axis=1).astype(jnp.int32)
    xc = jax.lax.dot_general(E, x, (((1,), (0,)), ((), ())),
                             preferred_element_type=jnp.float32)
    sqc = jnp.sum(E * sq[None, :], axis=1)
    prod = jax.lax.dot_general(xc, x, (((1,), (1,)), ((), ())),
                               preferred_element_type=jnp.float32)
    d2 = sqc[:, None] + sq[None, :] - 2.0 * prod
    distc = jnp.sqrt(jnp.maximum(d2, 0.0)) / _SQRT_C
    minv = jnp.min(distc, axis=0, keepdims=True)
    kio = jax.lax.broadcasted_iota(jnp.int32, (_CN, _N), 0)
    ic = jnp.min(jnp.where(distc == minv, kio, _CN), axis=0)
    ic_ref[0, 0] = jnp.where(rank < _CN, rank, ic)


def _merge_gather_kernel(x_ref, ic_ref, it_ref, aw_ref, xm_ref, itn_ref, awn_ref):
    ic = ic_ref[0, 0]
    kio = jax.lax.broadcasted_iota(jnp.int32, (_CN, _N), 0)
    A = (ic[None, :] == kio).astype(jnp.float32)
    count = jnp.sum(A, axis=1)
    inv = 1.0 / (count + 1e-06)
    nw = jnp.sum(A * inv[:, None], axis=0)
    xw = x_ref[0] * nw[:, None]
    xm_ref[0] = jax.lax.dot_general(A, xw, (((1,), (0,)), ((), ())),
                                    preferred_element_type=jnp.float32)
    it = it_ref[0, 0]
    mio = jax.lax.broadcasted_iota(jnp.int32, (_N, _N), 1)
    G = it[:, None] == mio
    itn_ref[0, 0] = jnp.sum(jnp.where(G, ic.astype(jnp.float32)[None, :], 0.0),
                            axis=1).astype(jnp.int32)
    wt = jnp.sum(jnp.where(G, nw[None, :], 0.0), axis=1)
    awn_ref[0, 0] = aw_ref[0, 0] * wt


def kernel(x, idx_token, agg_token, agg_weight):
    if agg_weight is None:
        agg_weight = agg_token
    x = x.astype(jnp.float32)
    sq = jnp.sum(x * x, axis=-1)
    sq3 = sq.reshape(_B, 1, _N)

    dist, dn, dmax = pl.pallas_call(
        _dist_stats_kernel,
        grid=(_B, _RT),
        in_specs=[
            pl.BlockSpec((1, _TM, _C), lambda b, t: (b, t, 0)),
            pl.BlockSpec((1, _N, _C), lambda b, t: (b, 0, 0)),
            pl.BlockSpec((1, 1, _TM), lambda b, t: (b, 0, t)),
            pl.BlockSpec((1, 1, _N), lambda b, t: (b, 0, 0)),
        ],
        out_specs=[
            pl.BlockSpec((1, _TM, _N), lambda b, t: (b, t, 0)),
            pl.BlockSpec((1, _K, _TM), lambda b, t: (b, 0, t)),
            pl.BlockSpec((1, 1, _TM), lambda b, t: (b, 0, t)),
        ],
        out_shape=[
            jax.ShapeDtypeStruct((_B, _N, _N), jnp.float32),
            jax.ShapeDtypeStruct((_B, _K, _N), jnp.float32),
            jax.ShapeDtypeStruct((_B, 1, _N), jnp.float32),
        ],
    )(x, x, sq3, sq3)

    dn_t = jnp.transpose(dn, (0, 2, 1))
    dens_flat = jnp.exp(-(dn_t ** 2).mean(axis=-1))
    dens_flat = dens_flat + jax.random.uniform(
        jax.random.key(1), dens_flat.shape, dtype=dens_flat.dtype) * 1e-06
    dens = dens_flat.reshape(_B, 1, _N)

    score = pl.pallas_call(
        _score_kernel,
        grid=(_B, _RT),
        in_specs=[
            pl.BlockSpec((1, _TM, _N), lambda b, t: (b, t, 0)),
            pl.BlockSpec((1, 1, _TM), lambda b, t: (b, 0, t)),
            pl.BlockSpec((1, 1, _N), lambda b, t: (b, 0, 0)),
            pl.BlockSpec((1, 1, _TM), lambda b, t: (b, 0, t)),
        ],
        out_specs=pl.BlockSpec((1, 1, _TM), lambda b, t: (b, 0, t)),
        out_shape=jax.ShapeDtypeStruct((_B, 1, _N), jnp.float32),
    )(dist, dens, dens, dmax)

    rank = pl.pallas_call(
        _rank_kernel,
        grid=(_B, _RT),
        in_specs=[
            pl.BlockSpec((1, 1, _N), lambda b, t: (b, 0, 0)),
            pl.BlockSpec((1, 1, _TM), lambda b, t: (b, 0, t)),
        ],
        out_specs=pl.BlockSpec((1, 1, _TM), lambda b, t: (b, 0, t)),
        out_shape=jax.ShapeDtypeStruct((_B, 1, _N), jnp.int32),
    )(score, score)

    idown, icl = pl.pallas_call(
        _centers_assign_kernel,
        grid=(_B,),
        in_specs=[
            pl.BlockSpec((1, 1, _N), lambda b: (b, 0, 0)),
            pl.BlockSpec((1, _N, _C), lambda b: (b, 0, 0)),
            pl.BlockSpec((1, 1, _N), lambda b: (b, 0, 0)),
        ],
        out_specs=[
            pl.BlockSpec((1, 1, _CN), lambda b: (b, 0, 0)),
            pl.BlockSpec((1, 1, _N), lambda b: (b, 0, 0)),
        ],
        out_shape=[
            jax.ShapeDtypeStruct((_B, 1, _CN), jnp.int32),
            jax.ShapeDtypeStruct((_B, 1, _N), jnp.int32),
        ],
    )(rank, x, sq3)

    it3 = idx_token.reshape(_B, 1, _N)
    aw3 = agg_weight.astype(jnp.float32).reshape(_B, 1, _N)
    xm, itn, awn = pl.pallas_call(
        _merge_gather_kernel,
        grid=(_B,),
        in_specs=[
            pl.BlockSpec((1, _N, _C), lambda b: (b, 0, 0)),
            pl.BlockSpec((1, 1, _N), lambda b: (b, 0, 0)),
            pl.BlockSpec((1, 1, _N), lambda b: (b, 0, 0)),
            pl.BlockSpec((1, 1, _N), lambda b: (b, 0, 0)),
        ],
        out_specs=[
            pl.BlockSpec((1, _CN, _C), lambda b: (b, 0, 0)),
            pl.BlockSpec((1, 1, _N), lambda b: (b, 0, 0)),
            pl.BlockSpec((1, 1, _N), lambda b: (b, 0, 0)),
        ],
        out_shape=[
            jax.ShapeDtypeStruct((_B, _CN, _C), jnp.float32),
            jax.ShapeDtypeStruct((_B, 1, _N), jnp.int32),
            jax.ShapeDtypeStruct((_B, 1, _N), jnp.float32),
        ],
    )(x, icl, it3, aw3)

    return (xm, itn.reshape(_B, _N), awn.reshape(_B, _N, 1),
            icl.reshape(_B, _N), idown.reshape(_B, _CN))
